# baseline (speedup 1.0000x reference)
import functools

import jax
import jax.numpy as jnp
from jax import lax
from jax.experimental import pallas as pl
from jax.experimental.pallas import tpu as pltpu
from jax.experimental.pallas import tpu_sc as plsc

B = 16384
E = 16
NC = 2
NS = 16
NW = NC * NS
BPW = B // NW
CH = 128
NCH = BPW // CH


def _sc_gather_dot(u_idx2d, r_idx2d, u_tabT, r_tabT, u_bias, r_bias):
    mesh = plsc.VectorSubcoreMesh(core_axis_name="c", subcore_axis_name="s")

    @functools.partial(
        pl.kernel,
        mesh=mesh,
        out_type=(
            jax.ShapeDtypeStruct((NW * E,), jnp.float32),
            jax.ShapeDtypeStruct((B,), jnp.float32),
        ),
        scratch_types=[
            pltpu.VMEM((NCH, CH), jnp.int32),
            pltpu.VMEM((NCH, CH), jnp.int32),
            pltpu.VMEM((E, BPW), jnp.float32),
            pltpu.VMEM((E, BPW), jnp.float32),
            pltpu.VMEM((BPW,), jnp.float32),
            pltpu.VMEM((BPW,), jnp.float32),
            pltpu.VMEM((BPW,), jnp.float32),
            pltpu.VMEM((E,), jnp.float32),
            pltpu.SemaphoreType.DMA,
        ],
        compiler_params=pltpu.CompilerParams(use_tc_tiling_on_sc=False),
    )
    def k(u_idx_hbm, r_idx_hbm, u_tab_hbm, r_tab_hbm, u_bias_hbm, r_bias_hbm,
          partial_hbm, ubrb_hbm, idx_u, idx_r, u_vals, r_vals, ub_v, rb_v,
          ubrb_v, acc_v, sem):
        wid = lax.axis_index("s") * NC + lax.axis_index("c")
        base = pl.multiple_of(wid * BPW, 8)
        row0 = wid * NCH

        pltpu.sync_copy(u_idx_hbm.at[pl.ds(row0, NCH)], idx_u)
        pltpu.sync_copy(r_idx_hbm.at[pl.ds(row0, NCH)], idx_r)

        for j in range(NCH):
            sl = pl.ds(j * CH, CH)
            copies = []
            for l in range(E):
                copies.append(pltpu.async_copy(
                    u_tab_hbm.at[l].at[idx_u.at[j]], u_vals.at[l, sl], sem))
                copies.append(pltpu.async_copy(
                    r_tab_hbm.at[l].at[idx_r.at[j]], r_vals.at[l, sl], sem))
            copies.append(pltpu.async_copy(u_bias_hbm.at[idx_u.at[j]], ub_v.at[sl], sem))
            copies.append(pltpu.async_copy(r_bias_hbm.at[idx_r.at[j]], rb_v.at[sl], sem))
            for c in copies:
                c.wait()

        acc = jnp.zeros((E,), jnp.float32)
        for i in range(BPW // E):
            sl = pl.ds(i * E, E)
            for l in range(E):
                acc = acc + u_vals[l, sl] * r_vals[l, sl]
        acc_v[...] = acc
        pltpu.sync_copy(acc_v, partial_hbm.at[pl.ds(pl.multiple_of(wid * E, 8), E)])

        for i in range(BPW // E):
            sl = pl.ds(i * E, E)
            ubrb_v[sl] = ub_v[sl] + rb_v[sl]
        pltpu.sync_copy(ubrb_v, ubrb_hbm.at[pl.ds(base, BPW)])

    return k(u_idx2d, r_idx2d, u_tabT, r_tabT, u_bias, r_bias)


def _tc_finish(partials_2d, ubrb_2d):
    def body(p_ref, x_ref, o_ref):
        s = jnp.sum(p_ref[...])
        o_ref[...] = jax.nn.sigmoid(x_ref[...] + s)

    return pl.pallas_call(
        body,
        out_shape=jax.ShapeDtypeStruct(ubrb_2d.shape, jnp.float32),
    )(partials_2d, ubrb_2d)


def kernel(inputs, user_embedding, user_bias, resto_embedding, resto_bias):
    idx = inputs.astype(jnp.int32)
    u_idx2d = idx[:, 0].reshape(B // CH, CH)
    r_idx2d = idx[:, 1].reshape(B // CH, CH)
    partials, ubrb = _sc_gather_dot(
        u_idx2d, r_idx2d, user_embedding.T, resto_embedding.T,
        user_bias.reshape(-1), resto_bias.reshape(-1))
    out = _tc_finish(partials.reshape(NW * E // 128, 128), ubrb.reshape(B // 128, 128))
    return out.reshape(B, 1)


# speedup vs baseline: 3.2557x; 3.2557x over previous
import functools

import jax
import jax.numpy as jnp
from jax import lax
from jax.experimental import pallas as pl
from jax.experimental.pallas import tpu as pltpu
from jax.experimental.pallas import tpu_sc as plsc

B = 16384
E = 16
NC = 2
NS = 16
NW = NC * NS
BPW = B // NW
CH = 128
NCH = BPW // CH
G = 125000


def _sc_dot(g_u2d, g_r2d, s_u2d, s_r2d, u_tab, r_tab):
    mesh = plsc.VectorSubcoreMesh(core_axis_name="c", subcore_axis_name="s")

    @functools.partial(
        pl.kernel,
        mesh=mesh,
        out_type=jax.ShapeDtypeStruct((NW * E,), jnp.float32),
        scratch_types=[
            pltpu.VMEM((NCH, CH), jnp.int32),
            pltpu.VMEM((NCH, CH), jnp.int32),
            pltpu.VMEM((NCH, CH), jnp.int32),
            pltpu.VMEM((NCH, CH), jnp.int32),
            pltpu.VMEM((2, CH, 128), jnp.float32),
            pltpu.VMEM((2, CH, 128), jnp.float32),
            pltpu.VMEM((E,), jnp.float32),
            pltpu.SemaphoreType.DMA,
            pltpu.SemaphoreType.DMA,
        ],
        compiler_params=pltpu.CompilerParams(needs_layout_passes=False),
    )
    def k(gu_hbm, gr_hbm, su_hbm, sr_hbm, u_tab_hbm, r_tab_hbm, partial_hbm,
          g_u, g_r, s_u, s_r, u_buf, r_buf, acc_v, sem0, sem1):
        wid = lax.axis_index("s") * NC + lax.axis_index("c")
        row0 = wid * NCH

        pltpu.sync_copy(gu_hbm.at[pl.ds(row0, NCH)], g_u)
        pltpu.sync_copy(gr_hbm.at[pl.ds(row0, NCH)], g_r)
        pltpu.sync_copy(su_hbm.at[pl.ds(row0, NCH)], s_u)
        pltpu.sync_copy(sr_hbm.at[pl.ds(row0, NCH)], s_r)

        sems = (sem0, sem1)

        def fire(j):
            bsel = j % 2
            return (
                pltpu.async_copy(u_tab_hbm.at[g_u.at[j]], u_buf.at[bsel], sems[bsel]),
                pltpu.async_copy(r_tab_hbm.at[g_r.at[j]], r_buf.at[bsel], sems[bsel]),
            )

        pend = fire(0)
        acc = jnp.zeros((E,), jnp.float32)
        for j in range(NCH):
            for c in pend:
                c.wait()
            if j + 1 < NCH:
                pend = fire(j + 1)
            bsel = j % 2
            for i in range(CH // E):
                rows = lax.iota(jnp.int32, E) + (i * E)
                sub_u = s_u[j, pl.ds(i * E, E)]
                sub_r = s_r[j, pl.ds(i * E, E)]
                for l in range(E):
                    uv = plsc.load_gather(u_buf.at[bsel], [rows, sub_u * E + l])
                    rv = plsc.load_gather(r_buf.at[bsel], [rows, sub_r * E + l])
                    acc = acc + uv * rv
        acc_v[...] = acc
        pltpu.sync_copy(acc_v, partial_hbm.at[pl.ds(pl.multiple_of(wid * E, 8), E)])

    return k(g_u2d, g_r2d, s_u2d, s_r2d, u_tab, r_tab)


def _sc_bias(u_idx2d, r_idx2d, u_bias, r_bias):
    mesh = plsc.VectorSubcoreMesh(core_axis_name="c", subcore_axis_name="s")

    @functools.partial(
        pl.kernel,
        mesh=mesh,
        out_type=jax.ShapeDtypeStruct((B,), jnp.float32),
        scratch_types=[
            pltpu.VMEM((NCH, CH), jnp.int32),
            pltpu.VMEM((NCH, CH), jnp.int32),
            pltpu.VMEM((BPW,), jnp.float32),
            pltpu.VMEM((BPW,), jnp.float32),
            pltpu.VMEM((BPW,), jnp.float32),
            pltpu.SemaphoreType.DMA,
        ],
        compiler_params=pltpu.CompilerParams(use_tc_tiling_on_sc=False),
    )
    def k(u_idx_hbm, r_idx_hbm, u_bias_hbm, r_bias_hbm, ubrb_hbm,
          idx_u, idx_r, ub_v, rb_v, ubrb_v, sem):
        wid = lax.axis_index("s") * NC + lax.axis_index("c")
        base = pl.multiple_of(wid * BPW, 8)
        row0 = wid * NCH

        pltpu.sync_copy(u_idx_hbm.at[pl.ds(row0, NCH)], idx_u)
        pltpu.sync_copy(r_idx_hbm.at[pl.ds(row0, NCH)], idx_r)

        copies = []
        for j in range(NCH):
            sl = pl.ds(j * CH, CH)
            copies.append(pltpu.async_copy(u_bias_hbm.at[idx_u.at[j]], ub_v.at[sl], sem))
            copies.append(pltpu.async_copy(r_bias_hbm.at[idx_r.at[j]], rb_v.at[sl], sem))
        for c in copies:
            c.wait()

        for i in range(BPW // E):
            sl = pl.ds(i * E, E)
            ubrb_v[sl] = ub_v[sl] + rb_v[sl]
        pltpu.sync_copy(ubrb_v, ubrb_hbm.at[pl.ds(base, BPW)])

    return k(u_idx2d, r_idx2d, u_bias, r_bias)


def _tc_finish(partials_2d, ubrb_2d):
    def body(p_ref, x_ref, o_ref):
        s = jnp.sum(p_ref[...])
        o_ref[...] = jax.nn.sigmoid(x_ref[...] + s)

    return pl.pallas_call(
        body,
        out_shape=jax.ShapeDtypeStruct(ubrb_2d.shape, jnp.float32),
    )(partials_2d, ubrb_2d)


def kernel(inputs, user_embedding, user_bias, resto_embedding, resto_bias):
    idx = inputs.astype(jnp.int32)
    u_idx = idx[:, 0]
    r_idx = idx[:, 1]
    shp = (B // CH, CH)
    partials = _sc_dot(
        (u_idx >> 3).reshape(shp), (r_idx >> 3).reshape(shp),
        (u_idx & 7).reshape(shp), (r_idx & 7).reshape(shp),
        user_embedding.reshape(G, 128), resto_embedding.reshape(G, 128))
    ubrb = _sc_bias(u_idx.reshape(shp), r_idx.reshape(shp),
                    user_bias.reshape(-1), resto_bias.reshape(-1))
    out = _tc_finish(partials.reshape(NW * E // 128, 128), ubrb.reshape(B // 128, 128))
    return out.reshape(B, 1)
